# baseline (speedup 1.0000x reference)
import functools

import jax
import jax.numpy as jnp
from jax import lax
from jax.experimental import pallas as pl
from jax.experimental.pallas import tpu as pltpu
from jax.experimental.pallas import tpu_sc as plsc

B, H, W = 16, 384, 384
PIX = H * W

K_SC = 4
K_TC = B - K_SC
IMGS = 4

NC, NS, L = 2, 16, 16
NW = NC * NS
CHUNK = (K_SC * PIX) // NW
CB = 4608
NB = CHUNK // CB
WPI = NW // K_SC

_C = (3.511021356650268e-06, 0.9997923620654495, -0.49697743071907685,
      0.31458917398920905, -0.1887808235491981, 0.08172564529133709,
      -0.01720779923058697)


def _sc_loss_kernel(x_hbm, p_hbm, n_hbm, label_hbm, out_hbm,
                    xv, pv, nv, lv, ov, sem0, sem1):
    wid = lax.axis_index("s") * NC + lax.axis_index("c")
    img = wid // WPI
    base = wid * CHUNK
    sems = (sem0, sem1)

    pltpu.sync_copy(label_hbm, lv)

    def _block_copies(g):
        slot = g % 2
        sem = sems[slot]
        off = base + g * CB
        return (
            pltpu.make_async_copy(x_hbm.at[pl.ds(off, CB)], xv.at[slot], sem),
            pltpu.make_async_copy(p_hbm.at[pl.ds(off, CB)], pv.at[slot], sem),
            pltpu.make_async_copy(n_hbm.at[pl.ds(off, CB)], nv.at[slot], sem),
        )

    for cp in _block_copies(0):
        cp.start()

    zeros = jnp.zeros((L,), jnp.float32)
    acc = (zeros, zeros, zeros, zeros)

    for g in range(NB):
        slot = g % 2
        if g + 1 < NB:
            for cp in _block_copies(g + 1):
                cp.start()
        for cp in _block_copies(g):
            cp.wait()

        def body(i, carry, slot=slot):
            a_log, a_max, a_x, a_cnt = carry
            x = xv[slot, pl.ds(i * L, L)]
            p = pv[slot, pl.ds(i * L, L)]
            n = nv[slot, pl.ds(i * L, L)]
            m = jnp.where((p > 0.5) & (n > 0.5), 1.0, 0.0)
            ax = jnp.abs(x)
            u = jnp.exp(-ax)
            q = jnp.float32(_C[6])
            for c in (_C[5], _C[4], _C[3], _C[2], _C[1], _C[0]):
                q = q * u + c
            a_log = a_log + m * q
            a_max = a_max + m * ((x + ax) * 0.5)
            a_x = a_x + m * x
            a_cnt = a_cnt + m
            return (a_log, a_max, a_x, a_cnt)

        acc = lax.fori_loop(0, CB // L, body, acc)

    s_log = jnp.sum(acc[0])
    s_max = jnp.sum(acc[1])
    s_x = jnp.sum(acc[2])
    s_cnt = jnp.sum(acc[3])
    idx = lax.broadcasted_iota(jnp.int32, (L,), 0)
    yv = lv[...].astype(jnp.float32)
    y = jnp.sum(jnp.where(idx == img, yv, 0.0))
    total = (1.0 + y) * (s_max + s_log) - (2.0 * y) * s_x

    vec = jnp.where(idx == 0, total, jnp.where(idx == 1, s_cnt, 0.0))
    ov[...] = vec
    pltpu.sync_copy(ov, out_hbm.at[wid])


_sc_loss = functools.partial(
    pl.kernel,
    mesh=plsc.VectorSubcoreMesh(core_axis_name="c", subcore_axis_name="s"),
    compiler_params=pltpu.CompilerParams(needs_layout_passes=False),
    out_type=jax.ShapeDtypeStruct((NW, L), jnp.float32),
    scratch_types=[
        pltpu.VMEM((2, CB), jnp.float32),
        pltpu.VMEM((2, CB), jnp.float32),
        pltpu.VMEM((2, CB), jnp.float32),
        pltpu.VMEM((B,), jnp.int32),
        pltpu.VMEM((L,), jnp.float32),
        pltpu.SemaphoreType.DMA,
        pltpu.SemaphoreType.DMA,
    ],
)(_sc_loss_kernel)


def _tc_loss_kernel(label_ref, x_ref, p_ref, n_ref, out_ref, acc_ref, cnt_ref):
    g = pl.program_id(0)

    @pl.when(g == 0)
    def _init():
        acc_ref[0] = 0.0
        cnt_ref[0] = 0.0

    total = 0.0
    count = 0.0
    for j in range(IMGS):
        x = x_ref[j]
        p = p_ref[j]
        n = n_ref[j]
        m = jnp.logical_and(p > 0.5, n > 0.5).astype(jnp.float32)
        y = label_ref[g * IMGS + j].astype(jnp.float32)
        u = jnp.exp(-jnp.abs(x))
        t = 1.0 + u * m
        v = t
        for _ in range(6):
            half = v.shape[0] // 2
            v = v[:half] * v[half:]
        s_log = jnp.sum(jnp.log(v))
        s_max = jnp.sum(m * jnp.maximum(x, 0.0))
        s_x = jnp.sum(m * x)
        total += (1.0 + y) * (s_max + s_log) - (2.0 * y) * s_x
        count += jnp.sum(m)
    acc_ref[0] += total
    cnt_ref[0] += count

    @pl.when(g == pl.num_programs(0) - 1)
    def _fini():
        out_ref[0] = acc_ref[0]
        out_ref[1] = cnt_ref[0]


def _tc_loss(label, x, p, n):
    grid_spec = pltpu.PrefetchScalarGridSpec(
        num_scalar_prefetch=1,
        grid=(K_TC // IMGS,),
        in_specs=[
            pl.BlockSpec((IMGS, H, W), lambda g, lbl: (g, 0, 0)),
            pl.BlockSpec((IMGS, H, W), lambda g, lbl: (g, 0, 0)),
            pl.BlockSpec((IMGS, H, W), lambda g, lbl: (g, 0, 0)),
        ],
        out_specs=pl.BlockSpec(memory_space=pltpu.SMEM),
        scratch_shapes=[
            pltpu.SMEM((1,), jnp.float32),
            pltpu.SMEM((1,), jnp.float32),
        ],
    )
    return pl.pallas_call(
        _tc_loss_kernel,
        grid_spec=grid_spec,
        out_shape=jax.ShapeDtypeStruct((2,), jnp.float32),
    )(label, x, p, n)


def kernel(cancer_logits, prostate_mask, needle_mask, label, involvement):
    x = cancer_logits.reshape(B, H, W)
    p = prostate_mask.reshape(B, H, W)
    n = needle_mask.reshape(B, H, W)
    label32 = label.astype(jnp.int32)

    sc_out = _sc_loss(
        x[:K_SC].reshape(-1), p[:K_SC].reshape(-1), n[:K_SC].reshape(-1),
        label32)
    tc_out = _tc_loss(label32[K_SC:], x[K_SC:], p[K_SC:], n[K_SC:])

    total = tc_out[0] + jnp.sum(sc_out[:, 0])
    count = tc_out[1] + jnp.sum(sc_out[:, 1])
    return total / count


# speedup vs baseline: 1.2630x; 1.2630x over previous
import functools

import jax
import jax.numpy as jnp
from jax import lax
from jax.experimental import pallas as pl
from jax.experimental.pallas import tpu as pltpu
from jax.experimental.pallas import tpu_sc as plsc

B, H, W = 16, 384, 384

K_SC = 4
K_TC = B - K_SC
IMGS = 4

NC, NS, L = 2, 16, 16
NW = NC * NS
WPI = NW // K_SC
RB = H // WPI
BR = 16
NB = RB // BR
CW = W // L

_C = (0.9998878719025601, -0.49636774398802214, 0.304670863083119,
      -0.15602693973930298, 0.0410640708360418)


def _sc_loss_kernel(x_hbm, p_hbm, n_hbm, label_hbm, out_hbm,
                    xv, pv, nv, lv, ov, sem0, sem1):
    wid = lax.axis_index("s") * NC + lax.axis_index("c")
    img = wid // WPI
    r0 = (wid % WPI) * RB
    sems = (sem0, sem1)

    pltpu.sync_copy(label_hbm, lv)

    def _block_copies(g):
        slot = g % 2
        sem = sems[slot]
        rows = pl.ds(r0 + g * BR, BR)
        return (
            pltpu.make_async_copy(x_hbm.at[img, 0, rows, :], xv.at[slot], sem),
            pltpu.make_async_copy(p_hbm.at[img, 0, rows, :], pv.at[slot], sem),
            pltpu.make_async_copy(n_hbm.at[img, 0, rows, :], nv.at[slot], sem),
        )

    for cp in _block_copies(0):
        cp.start()

    zeros = jnp.zeros((L,), jnp.float32)
    acc = (zeros, zeros, zeros, zeros)

    for g in range(NB):
        slot = g % 2
        if g + 1 < NB:
            for cp in _block_copies(g + 1):
                cp.start()
        for cp in _block_copies(g):
            cp.wait()

        def row_body(r, carry, slot=slot):
            a_log, a_x, a_absx, a_cnt = carry
            for c in range(CW):
                cols = pl.ds(c * L, L)
                x = xv[slot, r, cols]
                p = pv[slot, r, cols]
                n = nv[slot, r, cols]
                m = jnp.where((p > 0.5) & (n > 0.5), 1.0, 0.0)
                ax = jnp.abs(x)
                um = jnp.exp(-ax) * m
                g5 = jnp.float32(_C[4])
                for cf in (_C[3], _C[2], _C[1], _C[0]):
                    g5 = g5 * um + cf
                a_log = a_log + um * g5
                a_x = a_x + x * m
                a_absx = a_absx + ax * m
                a_cnt = a_cnt + m
            return (a_log, a_x, a_absx, a_cnt)

        acc = lax.fori_loop(0, BR, row_body, acc)

    s_log = jnp.sum(acc[0])
    s_x = jnp.sum(acc[1])
    s_max = (s_x + jnp.sum(acc[2])) * 0.5
    s_cnt = jnp.sum(acc[3])
    idx = lax.broadcasted_iota(jnp.int32, (L,), 0)
    yv = lv[...].astype(jnp.float32)
    y = jnp.sum(jnp.where(idx == img, yv, 0.0))
    total = (1.0 + y) * (s_max + s_log) - (2.0 * y) * s_x

    vec = jnp.where(idx == 0, total, jnp.where(idx == 1, s_cnt, 0.0))
    ov[...] = vec
    pltpu.sync_copy(ov, out_hbm.at[wid])


_sc_loss = functools.partial(
    pl.kernel,
    mesh=plsc.VectorSubcoreMesh(core_axis_name="c", subcore_axis_name="s"),
    compiler_params=pltpu.CompilerParams(needs_layout_passes=False),
    out_type=jax.ShapeDtypeStruct((NW, L), jnp.float32),
    scratch_types=[
        pltpu.VMEM((2, BR, W), jnp.float32),
        pltpu.VMEM((2, BR, W), jnp.float32),
        pltpu.VMEM((2, BR, W), jnp.float32),
        pltpu.VMEM((B,), jnp.int32),
        pltpu.VMEM((L,), jnp.float32),
        pltpu.SemaphoreType.DMA,
        pltpu.SemaphoreType.DMA,
    ],
)(_sc_loss_kernel)


def _tc_loss_kernel(label_ref, x_ref, p_ref, n_ref, out_ref, acc_ref, cnt_ref):
    g = pl.program_id(0)

    @pl.when(g == 0)
    def _init():
        acc_ref[0] = 0.0
        cnt_ref[0] = 0.0

    total = 0.0
    count = 0.0
    for j in range(IMGS):
        x = x_ref[j, 0]
        p = p_ref[j, 0]
        n = n_ref[j, 0]
        m = jnp.logical_and(p > 0.5, n > 0.5).astype(jnp.float32)
        y = label_ref[K_SC + g * IMGS + j].astype(jnp.float32)
        u = jnp.exp(-jnp.abs(x))
        t = 1.0 + u * m
        v = t
        for _ in range(6):
            half = v.shape[0] // 2
            v = v[:half] * v[half:]
        s_log = jnp.sum(jnp.log(v))
        s_max = jnp.sum(m * jnp.maximum(x, 0.0))
        s_x = jnp.sum(m * x)
        total += (1.0 + y) * (s_max + s_log) - (2.0 * y) * s_x
        count += jnp.sum(m)
    acc_ref[0] += total
    cnt_ref[0] += count

    @pl.when(g == pl.num_programs(0) - 1)
    def _fini():
        out_ref[0] = acc_ref[0]
        out_ref[1] = cnt_ref[0]


def _tc_loss(label, x, p, n):
    noff = K_SC // IMGS
    grid_spec = pltpu.PrefetchScalarGridSpec(
        num_scalar_prefetch=1,
        grid=(K_TC // IMGS,),
        in_specs=[
            pl.BlockSpec((IMGS, 1, H, W), lambda g, lbl: (g + noff, 0, 0, 0)),
            pl.BlockSpec((IMGS, 1, H, W), lambda g, lbl: (g + noff, 0, 0, 0)),
            pl.BlockSpec((IMGS, 1, H, W), lambda g, lbl: (g + noff, 0, 0, 0)),
        ],
        out_specs=pl.BlockSpec(memory_space=pltpu.SMEM),
        scratch_shapes=[
            pltpu.SMEM((1,), jnp.float32),
            pltpu.SMEM((1,), jnp.float32),
        ],
    )
    return pl.pallas_call(
        _tc_loss_kernel,
        grid_spec=grid_spec,
        out_shape=jax.ShapeDtypeStruct((2,), jnp.float32),
    )(label, x, p, n)


def kernel(cancer_logits, prostate_mask, needle_mask, label, involvement):
    label32 = label.astype(jnp.int32)

    sc_out = _sc_loss(cancer_logits, prostate_mask, needle_mask, label32)
    tc_out = _tc_loss(label32, cancer_logits, prostate_mask, needle_mask)

    total = tc_out[0] + jnp.sum(sc_out[:, 0])
    count = tc_out[1] + jnp.sum(sc_out[:, 1])
    return total / count


# speedup vs baseline: 3.5675x; 2.8247x over previous
import jax
import jax.numpy as jnp
from jax.experimental import pallas as pl
from jax.experimental.pallas import tpu as pltpu

B, H, W = 16, 384, 384
IMGS = 4

_C = (0.9998878719025601, -0.49636774398802214, 0.304670863083119,
      -0.15602693973930298, 0.0410640708360418)


def _loss_kernel(label_ref, x_ref, p_ref, n_ref, out_ref, acc_ref, cnt_ref):
    g = pl.program_id(0)

    @pl.when(g == 0)
    def _init():
        acc_ref[0] = 0.0
        cnt_ref[0] = 0.0

    total = 0.0
    count = 0.0
    for j in range(IMGS):
        x = x_ref[j, 0]
        p = p_ref[j, 0]
        n = n_ref[j, 0]
        m = jnp.logical_and(p > 0.5, n > 0.5).astype(jnp.float32)
        y = label_ref[g * IMGS + j].astype(jnp.float32)
        ax = jnp.abs(x)
        um = jnp.exp(-ax) * m
        gp = jnp.float32(_C[4])
        for cf in (_C[3], _C[2], _C[1], _C[0]):
            gp = gp * um + cf
        s_log = jnp.sum(um * gp)
        s_x = jnp.sum(x * m)
        s_max = (s_x + jnp.sum(ax * m)) * 0.5
        total += (1.0 + y) * (s_max + s_log) - (2.0 * y) * s_x
        count += jnp.sum(m)
    acc_ref[0] += total
    cnt_ref[0] += count

    @pl.when(g == pl.num_programs(0) - 1)
    def _fini():
        out_ref[0] = acc_ref[0] / cnt_ref[0]


def kernel(cancer_logits, prostate_mask, needle_mask, label, involvement):
    grid_spec = pltpu.PrefetchScalarGridSpec(
        num_scalar_prefetch=1,
        grid=(B // IMGS,),
        in_specs=[
            pl.BlockSpec((IMGS, 1, H, W), lambda g, lbl: (g, 0, 0, 0)),
            pl.BlockSpec((IMGS, 1, H, W), lambda g, lbl: (g, 0, 0, 0)),
            pl.BlockSpec((IMGS, 1, H, W), lambda g, lbl: (g, 0, 0, 0)),
        ],
        out_specs=pl.BlockSpec(memory_space=pltpu.SMEM),
        scratch_shapes=[
            pltpu.SMEM((1,), jnp.float32),
            pltpu.SMEM((1,), jnp.float32),
        ],
    )
    out = pl.pallas_call(
        _loss_kernel,
        grid_spec=grid_spec,
        out_shape=jax.ShapeDtypeStruct((1,), jnp.float32),
    )(label.astype(jnp.int32), cancer_logits, prostate_mask, needle_mask)
    return out[0]
